# baseline (speedup 1.0000x reference)
import functools

import jax
import jax.numpy as jnp
from jax import lax
from jax.experimental import pallas as pl
from jax.experimental.pallas import tpu as pltpu
from jax.experimental.pallas import tpu_sc as plsc

L = 16
NC = 2
NS = 16
NW = NC * NS
B = 16384
D = 64
V = 100000
CH = 4096
NCHK = B // CH
UN = 8


def _sc_body(feat_hbm, lab_hbm, centers_hbm, out_hbm,
             row_v, f0_v, f1_v, lab_v, acc_v, rsem, fsem, lsem):
    wid = lax.axis_index("s") * NC + lax.axis_index("c")

    centers3 = centers_hbm.reshape(D // 8, 8, V)
    feat3 = feat_hbm.reshape(D // 8, 8, B)
    fbufs = (f0_v, f1_v)

    def row_copy(d):
        return pltpu.async_copy(centers3.at[d // 8, d % 8], row_v, rsem)

    def feat_copy(d, c, buf):
        return pltpu.async_copy(
            feat3.at[d // 8, d % 8, pl.ds(c * CH, CH)], fbufs[buf], fsem)

    def chunk_compute(cbase, buf, acc):
        fb = fbufs[buf]

        def step(i, accs):
            acc = list(accs)
            for u in range(UN):
                idx = lab_v[pl.ds(cbase + (UN * i + u) * L, L)]
                f = fb[pl.ds((UN * i + u) * L, L)]
                c = plsc.load_gather(row_v, [idx])
                dd = f - c
                acc[u] = acc[u] + dd * dd
            return tuple(acc)

        return lax.fori_loop(0, CH // (UN * L), step, acc)

    d1, d2 = wid, wid + NW
    rcopy = row_copy(d1)
    lcopy = pltpu.async_copy(lab_hbm, lab_v, lsem)
    fcopy = feat_copy(d1, 0, 0)
    lcopy.wait()

    zero = jnp.zeros((L,), jnp.float32)
    acc = (zero,) * UN
    for k, d in enumerate((d1, d2)):
        rcopy.wait()
        for c in range(NCHK):
            nxt_fcopy = None
            if c + 1 < NCHK:
                nxt_fcopy = feat_copy(d, c + 1, (c + 1) % 2)
            elif k == 0:
                nxt_fcopy = feat_copy(d2, 0, (c + 1) % 2)
            fcopy.wait()
            acc = chunk_compute(c * CH, c % 2, acc)
            if c == NCHK - 1 and k == 0:
                rcopy = row_copy(d2)
            fcopy = nxt_fcopy

    t = [acc[2 * j] + acc[2 * j + 1] for j in range(UN // 2)]
    while len(t) > 1:
        t = [t[2 * j] + t[2 * j + 1] for j in range(len(t) // 2)]
    acc_v[...] = t[0]
    pltpu.sync_copy(acc_v, out_hbm.at[pl.ds(wid * L, L)])


@functools.partial(
    pl.kernel,
    out_type=jax.ShapeDtypeStruct((NW * L,), jnp.float32),
    mesh=plsc.VectorSubcoreMesh(core_axis_name="c", subcore_axis_name="s"),
    compiler_params=pltpu.CompilerParams(use_tc_tiling_on_sc=True,
                                        needs_layout_passes=False,
                                        disable_bounds_checks=True,
                                        disable_semaphore_checks=True),
    scratch_types=[
        pltpu.VMEM((V,), jnp.float32),
        pltpu.VMEM((CH,), jnp.float32),
        pltpu.VMEM((CH,), jnp.float32),
        pltpu.VMEM((B,), jnp.int32),
        pltpu.VMEM((L,), jnp.float32),
        pltpu.SemaphoreType.DMA,
        pltpu.SemaphoreType.DMA,
        pltpu.SemaphoreType.DMA,
    ],
)
def _centerloss_partials(feat_hbm, lab_hbm, centers_hbm, out_hbm,
                         row_v, f0_v, f1_v, lab_v, acc_v, rsem, fsem, lsem):
    _sc_body(feat_hbm, lab_hbm, centers_hbm, out_hbm,
             row_v, f0_v, f1_v, lab_v, acc_v, rsem, fsem, lsem)


def kernel(feature, label, centers, batch_size):
    partials = _centerloss_partials(feature.T, label.astype(jnp.int32),
                                    centers.T)
    return jnp.sum(partials) / 2.0 / batch_size


# speedup vs baseline: 1.0104x; 1.0104x over previous
import functools

import jax
import jax.numpy as jnp
from jax import lax
from jax.experimental import pallas as pl
from jax.experimental.pallas import tpu as pltpu
from jax.experimental.pallas import tpu_sc as plsc

L = 16
NC = 2
NS = 16
NW = NC * NS
B = 16384
D = 64
V = 100000
CH = 4096
NCHK = B // CH
UN = 4


def _sc_body(feat_hbm, lab_hbm, centers_hbm, out_hbm,
             row_v, f0_v, f1_v, lab_v, acc_v, rsem, fsem, lsem):
    wid = lax.axis_index("s") * NC + lax.axis_index("c")

    centers3 = centers_hbm.reshape(D // 8, 8, V)
    feat3 = feat_hbm.reshape(D // 8, 8, B)
    fbufs = (f0_v, f1_v)

    def row_copy(d):
        return pltpu.async_copy(centers3.at[d // 8, d % 8], row_v, rsem)

    def feat_copy(d, c, buf):
        return pltpu.async_copy(
            feat3.at[d // 8, d % 8, pl.ds(c * CH, CH)], fbufs[buf], fsem)

    def chunk_compute(cbase, buf, acc):
        fb = fbufs[buf]

        @plsc.parallel_loop(0, CH // (UN * L), unroll=2, carry=tuple(acc))
        def step(i, accs):
            acc = list(accs)
            for u in range(UN):
                idx = lab_v[pl.ds(cbase + (UN * i + u) * L, L)]
                f = fb[pl.ds((UN * i + u) * L, L)]
                c = plsc.load_gather(row_v, [idx])
                dd = f - c
                acc[u] = acc[u] + dd * dd
            return tuple(acc)

        return step

    d1, d2 = wid, wid + NW
    rcopy = row_copy(d1)
    lcopy = pltpu.async_copy(lab_hbm, lab_v, lsem)
    fcopy = feat_copy(d1, 0, 0)
    lcopy.wait()

    zero = jnp.zeros((L,), jnp.float32)
    acc = (zero,) * UN
    for k, d in enumerate((d1, d2)):
        rcopy.wait()
        for c in range(NCHK):
            nxt_fcopy = None
            if c + 1 < NCHK:
                nxt_fcopy = feat_copy(d, c + 1, (c + 1) % 2)
            elif k == 0:
                nxt_fcopy = feat_copy(d2, 0, (c + 1) % 2)
            fcopy.wait()
            acc = chunk_compute(c * CH, c % 2, acc)
            if c == NCHK - 1 and k == 0:
                rcopy = row_copy(d2)
            fcopy = nxt_fcopy

    t = [acc[2 * j] + acc[2 * j + 1] for j in range(UN // 2)]
    while len(t) > 1:
        t = [t[2 * j] + t[2 * j + 1] for j in range(len(t) // 2)]
    acc_v[...] = t[0]
    pltpu.sync_copy(acc_v, out_hbm.at[pl.ds(wid * L, L)])


@functools.partial(
    pl.kernel,
    out_type=jax.ShapeDtypeStruct((NW * L,), jnp.float32),
    mesh=plsc.VectorSubcoreMesh(core_axis_name="c", subcore_axis_name="s"),
    compiler_params=pltpu.CompilerParams(use_tc_tiling_on_sc=True,
                                        needs_layout_passes=False,
                                        disable_bounds_checks=True,
                                        disable_semaphore_checks=True),
    scratch_types=[
        pltpu.VMEM((V,), jnp.float32),
        pltpu.VMEM((CH,), jnp.float32),
        pltpu.VMEM((CH,), jnp.float32),
        pltpu.VMEM((B,), jnp.int32),
        pltpu.VMEM((L,), jnp.float32),
        pltpu.SemaphoreType.DMA,
        pltpu.SemaphoreType.DMA,
        pltpu.SemaphoreType.DMA,
    ],
)
def _centerloss_partials(feat_hbm, lab_hbm, centers_hbm, out_hbm,
                         row_v, f0_v, f1_v, lab_v, acc_v, rsem, fsem, lsem):
    _sc_body(feat_hbm, lab_hbm, centers_hbm, out_hbm,
             row_v, f0_v, f1_v, lab_v, acc_v, rsem, fsem, lsem)


def kernel(feature, label, centers, batch_size):
    partials = _centerloss_partials(feature.T, label.astype(jnp.int32),
                                    centers.T)
    return jnp.sum(partials) / 2.0 / batch_size
